# baseline (speedup 1.0000x reference)
import functools

import jax
import jax.numpy as jnp
from jax import lax
from jax.experimental import pallas as pl
from jax.experimental.pallas import tpu as pltpu, tpu_sc as plsc

B, C, T, NV, KT = 4, 32, 12, 10000, 3
T1 = T - (KT - 1)
T2 = T1 - (KT - 1)
F = B * C * T1
RP = 32
NPART = (NV + RP - 1) // RP
NW = 32
NPASS = (NPART + NW - 1) // NW
NVP = NPART * RP
ECH = 1024
EB = 32
EPAD = 2048
PTRPAD = NPART + 1 + 31


def _spmm_body(z_hbm, rows_hbm, cols_hbm, vals_hbm, ptr_hbm, o_hbm,
               ptr_v, rows_v, cols_v, vals_v, buf0, buf1, acc_v, sem0, sem1):
    wid = lax.axis_index("s") * 2 + lax.axis_index("c")
    pltpu.sync_copy(ptr_hbm, ptr_v)

    def process(off, eb, estart, eend, vs, buf):
        valvs = []
        rr = []
        for h in range(EB // 16):
            eidx = lax.iota(jnp.int32, 16) + (eb + off + h * 16)
            ok = (eidx >= estart) & (eidx < eend)
            v16 = jnp.where(ok, vals_v[pl.ds(off + h * 16, 16)], 0.0)
            r16 = jnp.clip(rows_v[pl.ds(off + h * 16, 16)] - vs, 0, RP - 1)
            valvs += [jnp.full((16,), v16[i], jnp.float32) for i in range(16)]
            rr += [r16[i] for i in range(16)]

        @plsc.parallel_loop(0, F // 16, step=1, unroll=2)
        def jloop(j):
            sl = pl.ds(j * 16, 16)
            for h in range(EB // 16):
                prods = [valvs[h * 16 + i] * buf[h * 16 + i, sl]
                         for i in range(16)]
                for i in range(16):
                    plsc.addupdate(acc_v.at[rr[h * 16 + i], sl], prods[i])

    def part_body(ps, _):
        p = ps * NW + wid

        @pl.when(p < NPART)
        def _():
            pv = ptr_v[pl.ds(p, 16)]
            estart = pv[0]
            eend = pv[1]
            vs = p * RP
            e0 = (estart // 8) * 8
            ne = eend - e0
            nch = (ne + ECH - 1) // ECH

            @plsc.parallel_loop(0, RP, step=1, unroll=1)
            def zr(r):
                for j in range(F // 16):
                    acc_v[r, pl.ds(j * 16, 16)] = jnp.zeros((16,), jnp.float32)

            def ch_body(ch, _):
                eb = e0 + ch * ECH
                pltpu.sync_copy(rows_hbm.at[pl.ds(eb, ECH + EB)], rows_v)
                pltpu.sync_copy(cols_hbm.at[pl.ds(eb, ECH + EB)], cols_v)
                pltpu.sync_copy(vals_hbm.at[pl.ds(eb, ECH + EB)], vals_v)
                pltpu.async_copy(z_hbm.at[cols_v.at[pl.ds(0, EB)]], buf0, sem0)

                def pair(q, _):
                    o0 = q * 2 * EB
                    pltpu.async_copy(
                        z_hbm.at[cols_v.at[pl.ds(o0 + EB, EB)]], buf1, sem1)
                    pltpu.make_async_copy(
                        z_hbm.at[cols_v.at[pl.ds(0, EB)]], buf0, sem0).wait()
                    process(o0, eb, estart, eend, vs, buf0)
                    pltpu.async_copy(
                        z_hbm.at[cols_v.at[pl.ds(o0 + 2 * EB, EB)]], buf0, sem0)
                    pltpu.make_async_copy(
                        z_hbm.at[cols_v.at[pl.ds(0, EB)]], buf1, sem1).wait()
                    process(o0 + EB, eb, estart, eend, vs, buf1)
                    return 0
                lax.fori_loop(0, ECH // (2 * EB), pair, 0)
                pltpu.make_async_copy(
                    z_hbm.at[cols_v.at[pl.ds(0, EB)]], buf0, sem0).wait()
                return 0
            lax.fori_loop(0, nch, ch_body, 0)
            pltpu.sync_copy(acc_v, o_hbm.at[pl.ds(vs, RP)])
        return 0
    lax.fori_loop(0, NPASS, part_body, 0)


def _spmm(z, rows_p, cols_p, vals_p, ptr_p):
    mesh = plsc.VectorSubcoreMesh(core_axis_name="c", subcore_axis_name="s")
    return pl.kernel(
        _spmm_body, mesh=mesh,
        out_type=jax.ShapeDtypeStruct((NVP, F), jnp.float32),
        scratch_types=[
            pltpu.VMEM((PTRPAD,), jnp.int32),
            pltpu.VMEM((ECH + EB,), jnp.int32),
            pltpu.VMEM((ECH + EB,), jnp.int32),
            pltpu.VMEM((ECH + EB,), jnp.float32),
            pltpu.VMEM((EB, F), jnp.float32),
            pltpu.VMEM((EB, F), jnp.float32),
            pltpu.VMEM((RP, F), jnp.float32),
            pltpu.SemaphoreType.DMA,
            pltpu.SemaphoreType.DMA,
        ],
    )(z, rows_p, cols_p, vals_p, ptr_p)


def _conv1_glu_body(x_ref, w0_ref, w1_ref, w2_ref, b_ref, o_ref):
    w0 = w0_ref[...]
    w1 = w1_ref[...]
    w2 = w2_ref[...]
    bias = b_ref[...]
    for t in range(T1):
        x0 = x_ref[0, :, t, :]
        x1 = x_ref[0, :, t + 1, :]
        x2 = x_ref[0, :, t + 2, :]
        xc = (jnp.dot(w0, x0, preferred_element_type=jnp.float32)
              + jnp.dot(w1, x1, preferred_element_type=jnp.float32)
              + jnp.dot(w2, x2, preferred_element_type=jnp.float32)
              + bias)
        o_ref[0, :, t, :] = (xc[:C, :] + x2) * jax.nn.sigmoid(xc[C:, :])


_NVB = 1280


def _conv1_glu(x, conv1_w, conv1_b):
    w = [conv1_w[:, :, k, 0] for k in range(KT)]
    bias = conv1_b[:, None]
    wspec = pl.BlockSpec((2 * C, C), lambda b, v: (0, 0))
    nvb = (NV + _NVB - 1) // _NVB
    return pl.pallas_call(
        _conv1_glu_body,
        grid=(B, nvb),
        in_specs=[pl.BlockSpec((1, C, T, _NVB), lambda b, v: (b, 0, 0, v)),
                  wspec, wspec, wspec,
                  pl.BlockSpec((2 * C, 1), lambda b, v: (0, 0))],
        out_specs=pl.BlockSpec((1, C, T1, _NVB), lambda b, v: (b, 0, 0, v)),
        out_shape=jax.ShapeDtypeStruct((B, C, T1, NV), jnp.float32),
    )(x, w[0], w[1], w[2], bias)


_NVT = 2048


def _tail_body(o4_ref, x1_ref, gw_ref, gb_ref, w20_ref, w21_ref, w22_ref,
               b2_ref, out_ref):
    gw = gw_ref[...]
    gb = gb_ref[...]
    w20, w21, w22 = w20_ref[...], w21_ref[...], w22_ref[...]
    b2 = b2_ref[...]
    xr = []
    for t in range(T1):
        g = lax.dot_general(gw, o4_ref[0, t], (((0,), (1,)), ((), ())),
                            preferred_element_type=jnp.float32)
        xr.append(jnp.maximum(g + gb + x1_ref[0, :, t, :], 0.0))
    for t in range(T2):
        y = (jnp.dot(w20, xr[t], preferred_element_type=jnp.float32)
             + jnp.dot(w21, xr[t + 1], preferred_element_type=jnp.float32)
             + jnp.dot(w22, xr[t + 2], preferred_element_type=jnp.float32)
             + b2)
        out_ref[0, :, t, :] = jnp.maximum(y + xr[t + 2], 0.0)


def _tail(o4, x1, gcn_w, gcn_b, conv2_w, conv2_b):
    w2 = [conv2_w[:, :, k, 0] for k in range(KT)]
    cspec = pl.BlockSpec((C, C), lambda b, v: (0, 0))
    rspec = pl.BlockSpec((C, 1), lambda b, v: (0, 0))
    nvt = (NV + _NVT - 1) // _NVT
    return pl.pallas_call(
        _tail_body,
        grid=(B, nvt),
        in_specs=[pl.BlockSpec((1, T1, _NVT, C), lambda b, v: (b, 0, v, 0)),
                  pl.BlockSpec((1, C, T1, _NVT), lambda b, v: (b, 0, 0, v)),
                  cspec, rspec, cspec, cspec, cspec, rspec],
        out_specs=pl.BlockSpec((1, C, T2, _NVT), lambda b, v: (b, 0, 0, v)),
        out_shape=jax.ShapeDtypeStruct((B, C, T2, NV), jnp.float32),
    )(o4, x1, gcn_w, gcn_b[:, None], w2[0], w2[1], w2[2], conv2_b[:, None])


def _ln_body(y_ref, gma_ref, bta_ref, out_ref):
    gma = gma_ref[...]
    bta = bta_ref[...]
    for t in range(T2):
        y = y_ref[0, :, t, :]
        mean = jnp.mean(y)
        var = jnp.mean((y - mean) ** 2)
        out_ref[0, :, t, :] = ((y - mean) * lax.rsqrt(var + 1e-5) * gma
                               + bta)


def _layernorm(y, gma_t, bta_t):
    gspec = pl.BlockSpec((C, NV), lambda b: (0, 0))
    return pl.pallas_call(
        _ln_body,
        grid=(B,),
        in_specs=[pl.BlockSpec((1, C, T2, NV), lambda b: (b, 0, 0, 0)),
                  gspec, gspec],
        out_specs=pl.BlockSpec((1, C, T2, NV), lambda b: (b, 0, 0, 0)),
        out_shape=jax.ShapeDtypeStruct((B, C, T2, NV), jnp.float32),
    )(y, gma_t, bta_t)


def kernel(x, conv1_w, conv1_b, gcn_w, gcn_b, conv2_w, conv2_b,
           ln_gamma, ln_beta, filter_vals, filter_rows, filter_cols):
    order = jnp.argsort(filter_rows)
    rows_s = filter_rows[order]
    cols_s = filter_cols[order]
    vals_s = filter_vals[order]
    bounds = jnp.arange(NPART + 1, dtype=jnp.int32) * RP
    ptr = jnp.searchsorted(rows_s, bounds, side="left").astype(jnp.int32)
    ptr_p = jnp.pad(ptr, (0, PTRPAD - NPART - 1))
    rows_p = jnp.pad(rows_s, (0, EPAD))
    cols_p = jnp.pad(cols_s, (0, EPAD))
    vals_p = jnp.pad(vals_s, (0, EPAD))

    x1 = _conv1_glu(x, conv1_w, conv1_b)
    z = x1.reshape(NV, F)
    o = _spmm(z, rows_p, cols_p, vals_p, ptr_p)
    o4 = o[:NV].reshape(B, T1, NV, C)
    y = _tail(o4, x1, gcn_w, gcn_b, conv2_w, conv2_b)
    return _layernorm(y, ln_gamma.T, ln_beta.T)


# speedup vs baseline: 1.3644x; 1.3644x over previous
import jax
import jax.numpy as jnp
from jax import lax
from jax.experimental import pallas as pl
from jax.experimental.pallas import tpu as pltpu, tpu_sc as plsc

B, C, T, NV, KT = 4, 32, 12, 10000, 3
T1 = T - (KT - 1)
T2 = T1 - (KT - 1)
F = B * C * T1
RP = 40
NPART = (NV + RP - 1) // RP
NW = 32
NPASS = (NPART + NW - 1) // NW
NVP = NPART * RP
ECH = 1024
EB = 16
EPAD = 2048
PTRPAD = NPART + 1 + 31


def _spmm_body(z_hbm, rows_hbm, cols_hbm, vals_hbm, ptr_hbm, o_hbm,
               ptr_v, rows_v, cols_v, vals_v, buf0, buf1, acc_v, sem0, sem1):
    wid = lax.axis_index("s") * 2 + lax.axis_index("c")
    pltpu.sync_copy(ptr_hbm, ptr_v)

    def process(off, eb, estart, eend, vs, buf):
        eidx = lax.iota(jnp.int32, EB) + (eb + off)
        ok = (eidx >= estart) & (eidx < eend)
        v16 = jnp.where(ok, vals_v[pl.ds(off, EB)], 0.0)
        r16 = jnp.clip(rows_v[pl.ds(off, EB)] - vs, 0, RP - 1)
        valvs = [jnp.full((16,), v16[i], jnp.float32) for i in range(EB)]
        rr = [r16[i] for i in range(EB)]

        @plsc.parallel_loop(0, F // 16, step=1, unroll=4)
        def jloop(j):
            sl = pl.ds(j * 16, 16)
            prods = [valvs[i] * buf[i, sl] for i in range(EB)]
            for i in range(EB):
                plsc.addupdate(acc_v.at[rr[i], sl], prods[i])

    def part_body(ps, _):
        p = ps * NW + wid

        @pl.when(p < NPART)
        def _():
            pv = ptr_v[pl.ds(p, 16)]
            estart = pv[0]
            eend = pv[1]
            vs = p * RP
            e0 = (estart // 8) * 8
            ne = eend - e0
            nch = (ne + ECH - 1) // ECH

            @plsc.parallel_loop(0, RP, step=1, unroll=1)
            def zr(r):
                for j in range(F // 16):
                    acc_v[r, pl.ds(j * 16, 16)] = jnp.zeros((16,), jnp.float32)

            def ch_body(ch, _):
                eb = e0 + ch * ECH
                pltpu.sync_copy(rows_hbm.at[pl.ds(eb, ECH + EB)], rows_v)
                pltpu.sync_copy(cols_hbm.at[pl.ds(eb, ECH + EB)], cols_v)
                pltpu.sync_copy(vals_hbm.at[pl.ds(eb, ECH + EB)], vals_v)
                pltpu.async_copy(z_hbm.at[cols_v.at[pl.ds(0, EB)]], buf0, sem0)

                def pair(q, _):
                    o0 = q * 2 * EB
                    pltpu.async_copy(
                        z_hbm.at[cols_v.at[pl.ds(o0 + EB, EB)]], buf1, sem1)
                    pltpu.make_async_copy(
                        z_hbm.at[cols_v.at[pl.ds(0, EB)]], buf0, sem0).wait()
                    process(o0, eb, estart, eend, vs, buf0)
                    pltpu.async_copy(
                        z_hbm.at[cols_v.at[pl.ds(o0 + 2 * EB, EB)]], buf0, sem0)
                    pltpu.make_async_copy(
                        z_hbm.at[cols_v.at[pl.ds(0, EB)]], buf1, sem1).wait()
                    process(o0 + EB, eb, estart, eend, vs, buf1)
                    return 0
                lax.fori_loop(0, ECH // (2 * EB), pair, 0)
                pltpu.make_async_copy(
                    z_hbm.at[cols_v.at[pl.ds(0, EB)]], buf0, sem0).wait()
                return 0
            lax.fori_loop(0, nch, ch_body, 0)
            pltpu.sync_copy(acc_v, o_hbm.at[pl.ds(vs, RP)])
        return 0
    lax.fori_loop(0, NPASS, part_body, 0)


def _spmm(z, rows_p, cols_p, vals_p, ptr_p):
    mesh = plsc.VectorSubcoreMesh(core_axis_name="c", subcore_axis_name="s")
    return pl.kernel(
        _spmm_body, mesh=mesh,
        out_type=jax.ShapeDtypeStruct((NVP, F), jnp.float32),
        scratch_types=[
            pltpu.VMEM((PTRPAD,), jnp.int32),
            pltpu.VMEM((ECH + EB,), jnp.int32),
            pltpu.VMEM((ECH + EB,), jnp.int32),
            pltpu.VMEM((ECH + EB,), jnp.float32),
            pltpu.VMEM((EB, F), jnp.float32),
            pltpu.VMEM((EB, F), jnp.float32),
            pltpu.VMEM((RP, F), jnp.float32),
            pltpu.SemaphoreType.DMA,
            pltpu.SemaphoreType.DMA,
        ],
    )(z, rows_p, cols_p, vals_p, ptr_p)


def _conv1_glu_body(x_ref, w0_ref, w1_ref, w2_ref, b_ref, o_ref):
    w0 = w0_ref[...]
    w1 = w1_ref[...]
    w2 = w2_ref[...]
    bias = b_ref[...]
    for t in range(T1):
        x0 = x_ref[0, :, t, :]
        x1 = x_ref[0, :, t + 1, :]
        x2 = x_ref[0, :, t + 2, :]
        xc = (jnp.dot(w0, x0, preferred_element_type=jnp.float32)
              + jnp.dot(w1, x1, preferred_element_type=jnp.float32)
              + jnp.dot(w2, x2, preferred_element_type=jnp.float32)
              + bias)
        o_ref[0, :, t, :] = (xc[:C, :] + x2) * jax.nn.sigmoid(xc[C:, :])


_NVB = 1280


def _conv1_glu(x, conv1_w, conv1_b):
    w = [conv1_w[:, :, k, 0] for k in range(KT)]
    bias = conv1_b[:, None]
    wspec = pl.BlockSpec((2 * C, C), lambda b, v: (0, 0))
    nvb = (NV + _NVB - 1) // _NVB
    return pl.pallas_call(
        _conv1_glu_body,
        grid=(B, nvb),
        in_specs=[pl.BlockSpec((1, C, T, _NVB), lambda b, v: (b, 0, 0, v)),
                  wspec, wspec, wspec,
                  pl.BlockSpec((2 * C, 1), lambda b, v: (0, 0))],
        out_specs=pl.BlockSpec((1, C, T1, _NVB), lambda b, v: (b, 0, 0, v)),
        out_shape=jax.ShapeDtypeStruct((B, C, T1, NV), jnp.float32),
    )(x, w[0], w[1], w[2], bias)


_NVT = 2048


def _tail_body(o4_ref, x1_ref, gw_ref, gb_ref, w20_ref, w21_ref, w22_ref,
               b2_ref, out_ref):
    gw = gw_ref[...]
    gb = gb_ref[...]
    w20, w21, w22 = w20_ref[...], w21_ref[...], w22_ref[...]
    b2 = b2_ref[...]
    xr = []
    for t in range(T1):
        g = lax.dot_general(gw, o4_ref[0, t], (((0,), (1,)), ((), ())),
                            preferred_element_type=jnp.float32)
        xr.append(jnp.maximum(g + gb + x1_ref[0, :, t, :], 0.0))
    for t in range(T2):
        y = (jnp.dot(w20, xr[t], preferred_element_type=jnp.float32)
             + jnp.dot(w21, xr[t + 1], preferred_element_type=jnp.float32)
             + jnp.dot(w22, xr[t + 2], preferred_element_type=jnp.float32)
             + b2)
        out_ref[0, :, t, :] = jnp.maximum(y + xr[t + 2], 0.0)


def _tail(o4, x1, gcn_w, gcn_b, conv2_w, conv2_b):
    w2 = [conv2_w[:, :, k, 0] for k in range(KT)]
    cspec = pl.BlockSpec((C, C), lambda b, v: (0, 0))
    rspec = pl.BlockSpec((C, 1), lambda b, v: (0, 0))
    nvt = (NV + _NVT - 1) // _NVT
    return pl.pallas_call(
        _tail_body,
        grid=(B, nvt),
        in_specs=[pl.BlockSpec((1, T1, _NVT, C), lambda b, v: (b, 0, v, 0)),
                  pl.BlockSpec((1, C, T1, _NVT), lambda b, v: (b, 0, 0, v)),
                  cspec, rspec, cspec, cspec, cspec, rspec],
        out_specs=pl.BlockSpec((1, C, T2, _NVT), lambda b, v: (b, 0, 0, v)),
        out_shape=jax.ShapeDtypeStruct((B, C, T2, NV), jnp.float32),
    )(o4, x1, gcn_w, gcn_b[:, None], w2[0], w2[1], w2[2], conv2_b[:, None])


def _ln_body(y_ref, gma_ref, bta_ref, out_ref):
    gma = gma_ref[...]
    bta = bta_ref[...]
    for t in range(T2):
        y = y_ref[0, :, t, :]
        mean = jnp.mean(y)
        var = jnp.mean((y - mean) ** 2)
        out_ref[0, :, t, :] = ((y - mean) * lax.rsqrt(var + 1e-5) * gma
                               + bta)


def _layernorm(y, gma_t, bta_t):
    gspec = pl.BlockSpec((C, NV), lambda b: (0, 0))
    return pl.pallas_call(
        _ln_body,
        grid=(B,),
        in_specs=[pl.BlockSpec((1, C, T2, NV), lambda b: (b, 0, 0, 0)),
                  gspec, gspec],
        out_specs=pl.BlockSpec((1, C, T2, NV), lambda b: (b, 0, 0, 0)),
        out_shape=jax.ShapeDtypeStruct((B, C, T2, NV), jnp.float32),
    )(y, gma_t, bta_t)


def kernel(x, conv1_w, conv1_b, gcn_w, gcn_b, conv2_w, conv2_b,
           ln_gamma, ln_beta, filter_vals, filter_rows, filter_cols):
    order = jnp.argsort(filter_rows)
    rows_s = filter_rows[order]
    cols_s = filter_cols[order]
    vals_s = filter_vals[order]
    bounds = jnp.arange(NPART + 1, dtype=jnp.int32) * RP
    ptr = jnp.searchsorted(rows_s, bounds, side="left").astype(jnp.int32)
    ptr_p = jnp.pad(ptr, (0, PTRPAD - NPART - 1))
    rows_p = jnp.pad(rows_s, (0, EPAD))
    cols_p = jnp.pad(cols_s, (0, EPAD))
    vals_p = jnp.pad(vals_s, (0, EPAD))

    x1 = _conv1_glu(x, conv1_w, conv1_b)
    z = x1.reshape(NV, F)
    o = _spmm(z, rows_p, cols_p, vals_p, ptr_p)
    o4 = o[:NV].reshape(B, T1, NV, C)
    y = _tail(o4, x1, gcn_w, gcn_b, conv2_w, conv2_b)
    return _layernorm(y, ln_gamma.T, ln_beta.T)
